# baseline (speedup 1.0000x reference)
import functools

import jax
import jax.numpy as jnp
from jax import lax
from jax.experimental import pallas as pl
from jax.experimental.pallas import tpu as pltpu
from jax.experimental.pallas import tpu_sc as plsc

N = 10000
D = 128
NC = 2
NS = 16
NW = NC * NS
N_PAD = 10240
DCHUNK = 128
CHUNK = 128
NBUF = 2
NGRP = 2
ROWS_PT = N_PAD // NS
BLK = 1024
GRID = N_PAD // BLK

_mesh = plsc.VectorSubcoreMesh(core_axis_name="c", subcore_axis_name="s",
                               num_cores=NC, num_subcores=NS)


def _make_deg_kernel(n_ch):
    K = 8
    n_grp = n_ch // K

    @functools.partial(
        pl.kernel,
        out_type=jax.ShapeDtypeStruct((NC * N_PAD,), jnp.float32),
        mesh=_mesh,
        scratch_types=[
            pltpu.VMEM_SHARED((N_PAD,), jnp.float32),
            pltpu.VMEM((n_ch, DCHUNK), jnp.int32),
            pltpu.VMEM((DCHUNK,), jnp.float32),
            pltpu.VMEM((ROWS_PT,), jnp.float32),
            pltpu.SemaphoreType.DMA,
        ],
    )
    def deg_kernel(dst_hbm, out_hbm, accum, dst_all, ones_v, zbuf, sem):
        c = lax.axis_index("c")
        s = lax.axis_index("s")
        wid = s * NC + c

        pltpu.sync_copy(dst_hbm.at[pl.ds(wid * n_ch, n_ch)], dst_all)
        zero16 = jnp.zeros((16,), jnp.float32)
        one16 = jnp.full((16,), 1.0, jnp.float32)

        def zb(i, _):
            zbuf[pl.ds(i * 16, 16)] = zero16
            return 0

        lax.fori_loop(0, ROWS_PT // 16, zb, 0)
        for j in range(DCHUNK // 16):
            ones_v[pl.ds(j * 16, 16)] = one16
        pltpu.sync_copy(zbuf, accum.at[pl.ds(s * ROWS_PT, ROWS_PT)])
        plsc.subcore_barrier()

        def grp(g, _):
            for k in range(K):
                pltpu.async_copy(ones_v, accum.at[dst_all.at[g * K + k]],
                                 sem, add=True)
            for k in range(K):
                pltpu.make_async_copy(ones_v, accum.at[dst_all.at[0]],
                                      sem).wait()
            return 0

        lax.fori_loop(0, n_grp, grp, 0)
        plsc.subcore_barrier()
        pltpu.sync_copy(accum.at[pl.ds(s * ROWS_PT, ROWS_PT)],
                        out_hbm.at[pl.ds(c * N_PAD + s * ROWS_PT, ROWS_PT)])

    return deg_kernel


def _make_edge_kernel(n_ch):
    G = n_ch // NGRP
    n_outer = G // NBUF

    @functools.partial(
        pl.kernel,
        out_type=jax.ShapeDtypeStruct((NC * N_PAD, D), jnp.float32),
        mesh=_mesh,
        scratch_types=[
            pltpu.VMEM_SHARED((N_PAD, D), jnp.float32),
            pltpu.VMEM((G, CHUNK), jnp.int32),
            pltpu.VMEM((G, CHUNK), jnp.int32),
            pltpu.VMEM((NBUF, CHUNK, D), jnp.float32),
        ] + [pltpu.SemaphoreType.DMA] * (2 * NBUF + 1),
    )
    def edge_kernel(hs_hbm, src_hbm, dst_hbm, out_hbm,
                    accum, src_g, dst_g, rows,
                    g0, g1, s0, s1, wsem):
        gsem = (g0, g1)
        ssem = (s0, s1)
        c = lax.axis_index("c")
        s = lax.axis_index("s")
        wid = s * NC + c

        zero16 = jnp.zeros((16,), jnp.float32)

        def zb(i, _):
            for j in range(D // 16):
                rows[0, i, pl.ds(j * 16, 16)] = zero16
            return 0

        lax.fori_loop(0, CHUNK, zb, 0)
        for k in range(ROWS_PT // CHUNK):
            pltpu.async_copy(rows.at[0],
                             accum.at[pl.ds(s * ROWS_PT + k * CHUNK, CHUNK)],
                             wsem)
        for k in range(ROWS_PT // CHUNK):
            pltpu.make_async_copy(rows.at[0], accum.at[pl.ds(0, CHUNK)],
                                  wsem).wait()
        plsc.subcore_barrier()

        def gfire(j, b):
            pltpu.async_copy(hs_hbm.at[src_g.at[j]], rows.at[b], gsem[b])

        def gwait(b):
            pltpu.make_async_copy(hs_hbm.at[src_g.at[0]], rows.at[b],
                                  gsem[b]).wait()

        def sfire(j, b):
            pltpu.async_copy(rows.at[b], accum.at[dst_g.at[j]], ssem[b],
                             add=True)

        def swait(b):
            pltpu.make_async_copy(rows.at[b], accum.at[dst_g.at[0]],
                                  ssem[b]).wait()

        def group(g, _):
            base = wid * n_ch + g * G
            pltpu.sync_copy(src_hbm.at[pl.ds(base, G)], src_g)
            pltpu.sync_copy(dst_hbm.at[pl.ds(base, G)], dst_g)

            for b in range(NBUF):
                gfire(b, b)

            def outer(t, _):
                for b in range(NBUF):
                    gwait(b)
                    sfire(t * NBUF + b, b)
                for b in range(NBUF):
                    swait(b)

                    @pl.when(t < n_outer - 1)
                    def _():
                        gfire((t + 1) * NBUF + b, b)

                return 0

            lax.fori_loop(0, n_outer, outer, 0)
            return 0

        lax.fori_loop(0, NGRP, group, 0)
        plsc.subcore_barrier()
        for k in range(ROWS_PT // CHUNK):
            r = s * ROWS_PT + k * CHUNK
            pltpu.async_copy(accum.at[pl.ds(r, CHUNK)],
                             out_hbm.at[pl.ds(c * N_PAD + r, CHUNK)], wsem)
        for k in range(ROWS_PT // CHUNK):
            pltpu.make_async_copy(accum.at[pl.ds(0, CHUNK)],
                                  out_hbm.at[pl.ds(0, CHUNK)], wsem).wait()

    return edge_kernel


def _dinv_body(deg_ref, out_ref):
    d = deg_ref[...]
    row = lax.rsqrt(d[0:1, :] + d[1:2, :] + 1.0)
    ri = lax.broadcasted_iota(jnp.int32, (BLK, BLK), 0)
    ci = lax.broadcasted_iota(jnp.int32, (BLK, BLK), 1)
    col = jnp.sum(jnp.where(ri == ci, row, 0.0), axis=1, keepdims=True)
    out_ref[...] = col


def _mm_body(x_ref, w_ref, out_ref):
    out_ref[...] = jnp.dot(x_ref[...], w_ref[...],
                           preferred_element_type=jnp.float32)


def _scale_body(h_ref, dinv_ref, out_ref):
    out_ref[...] = h_ref[...] * dinv_ref[...]


def _k2_body(s0_ref, s1_ref, hs_ref, dinv_ref, b_ref, w_ref, out_ref):
    dinv = dinv_ref[...]
    y = jnp.maximum(dinv * (s0_ref[...] + s1_ref[...] + hs_ref[...])
                    + b_ref[...], 0.0)
    out_ref[...] = jnp.dot(y, w_ref[...],
                           preferred_element_type=jnp.float32) * dinv


def _k3_body(s0_ref, s1_ref, hs_ref, dinv_ref, b_ref, w_ref, bfc_ref, out_ref):
    dinv = dinv_ref[...]
    y = jnp.maximum(dinv * (s0_ref[...] + s1_ref[...] + hs_ref[...])
                    + b_ref[...], 0.0)
    out_ref[...] = (jnp.dot(y, w_ref[...], preferred_element_type=jnp.float32)
                    + bfc_ref[...])


_row_spec = pl.BlockSpec((BLK, D), lambda i: (i, 0))
_dinv_spec = pl.BlockSpec((BLK, 1), lambda i: (i, 0))
_w_spec = pl.BlockSpec((D, D), lambda i: (0, 0))
_b_spec = pl.BlockSpec((1, D), lambda i: (0, 0))
_s0_spec = pl.BlockSpec((BLK, D), lambda i: (i, 0))
_s1_spec = pl.BlockSpec((BLK, D), lambda i: (GRID + i, 0))

_dinv_call = pl.pallas_call(
    _dinv_body,
    grid=(GRID,),
    in_specs=[pl.BlockSpec((2, BLK), lambda i: (0, i))],
    out_specs=_dinv_spec,
    out_shape=jax.ShapeDtypeStruct((N_PAD, 1), jnp.float32),
)

_mm_call = pl.pallas_call(
    _mm_body,
    grid=(GRID,),
    in_specs=[_row_spec, _w_spec],
    out_specs=_row_spec,
    out_shape=jax.ShapeDtypeStruct((N_PAD, D), jnp.float32),
)

_scale_call = pl.pallas_call(
    _scale_body,
    grid=(GRID,),
    in_specs=[_row_spec, _dinv_spec],
    out_specs=_row_spec,
    out_shape=jax.ShapeDtypeStruct((N_PAD, D), jnp.float32),
)

_k2_call = pl.pallas_call(
    _k2_body,
    grid=(GRID,),
    in_specs=[_s0_spec, _s1_spec, _row_spec, _dinv_spec, _b_spec, _w_spec],
    out_specs=_row_spec,
    out_shape=jax.ShapeDtypeStruct((N_PAD, D), jnp.float32),
)

_k3_call = pl.pallas_call(
    _k3_body,
    grid=(GRID,),
    in_specs=[_s0_spec, _s1_spec, _row_spec, _dinv_spec, _b_spec, _w_spec,
              _b_spec],
    out_specs=_row_spec,
    out_shape=jax.ShapeDtypeStruct((N_PAD, D), jnp.float32),
)


def kernel(x, edge_index, W1, b1, W2, b2, Wfc, bfc):
    E = edge_index.shape[1]
    grp = NGRP * NBUF
    n_ch = pl.cdiv(E, NW * CHUNK * grp) * grp
    e_pad = n_ch * CHUNK * NW
    n_ch_deg = e_pad // (NW * DCHUNK)
    pad = e_pad - E

    src_flat = jnp.concatenate([edge_index[0], jnp.zeros((pad,), jnp.int32)])
    dst_flat = jnp.concatenate(
        [edge_index[1], jnp.full((pad,), N_PAD - 1, jnp.int32)])
    src = src_flat.reshape(-1, CHUNK)
    dst = dst_flat.reshape(-1, CHUNK)
    dst_d = dst_flat.reshape(-1, DCHUNK)
    x_pad = jnp.pad(x, ((0, N_PAD - N), (0, 0)))

    deg_kernel = _make_deg_kernel(n_ch_deg)
    edge_kernel = _make_edge_kernel(n_ch)

    deg2 = deg_kernel(dst_d).reshape(2, N_PAD)
    h1 = _mm_call(x_pad, W1)
    dinv = _dinv_call(deg2)

    b1r = b1.reshape(1, D)
    b2r = b2.reshape(1, D)
    bfcr = bfc.reshape(1, D)

    hs1 = _scale_call(h1, dinv)
    S1 = edge_kernel(hs1, src, dst)
    hs2 = _k2_call(S1, S1, hs1, dinv, b1r, W2)
    S2 = edge_kernel(hs2, src, dst)
    out = _k3_call(S2, S2, hs2, dinv, b2r, Wfc, bfcr)
    return out[:N]


# speedup vs baseline: 1.1339x; 1.1339x over previous
import functools

import jax
import jax.numpy as jnp
from jax import lax
from jax.experimental import pallas as pl
from jax.experimental.pallas import tpu as pltpu
from jax.experimental.pallas import tpu_sc as plsc

N = 10000
D = 128
NC = 2
NS = 16
NW = NC * NS
N_PAD = 10240
DCHUNK = 128
CHUNK = 128
NBUF = 2
NGRP = 2
ROWS_PT = N_PAD // NS
BLK = 1024
GRID = N_PAD // BLK

_mesh = plsc.VectorSubcoreMesh(core_axis_name="c", subcore_axis_name="s",
                               num_cores=NC, num_subcores=NS)


def _make_deg_kernel(n_ch):
    K = 8
    n_grp = n_ch // K

    @functools.partial(
        pl.kernel,
        out_type=jax.ShapeDtypeStruct((NC * N_PAD,), jnp.float32),
        mesh=_mesh,
        scratch_types=[
            pltpu.VMEM_SHARED((N_PAD,), jnp.float32),
            pltpu.VMEM((n_ch, DCHUNK), jnp.int32),
            pltpu.VMEM((DCHUNK,), jnp.float32),
            pltpu.VMEM((ROWS_PT,), jnp.float32),
            pltpu.SemaphoreType.DMA,
        ],
    )
    def deg_kernel(dst_hbm, out_hbm, accum, dst_all, ones_v, zbuf, sem):
        c = lax.axis_index("c")
        s = lax.axis_index("s")
        wid = s * NC + c

        pltpu.sync_copy(dst_hbm.at[pl.ds(wid * n_ch, n_ch)], dst_all)
        zero16 = jnp.zeros((16,), jnp.float32)
        one16 = jnp.full((16,), 1.0, jnp.float32)

        def zb(i, _):
            zbuf[pl.ds(i * 16, 16)] = zero16
            return 0

        lax.fori_loop(0, ROWS_PT // 16, zb, 0)
        for j in range(DCHUNK // 16):
            ones_v[pl.ds(j * 16, 16)] = one16
        pltpu.sync_copy(zbuf, accum.at[pl.ds(s * ROWS_PT, ROWS_PT)])
        plsc.subcore_barrier()

        def grp(g, _):
            for k in range(K):
                pltpu.async_copy(ones_v, accum.at[dst_all.at[g * K + k]],
                                 sem, add=True)
            for k in range(K):
                pltpu.make_async_copy(ones_v, accum.at[dst_all.at[0]],
                                      sem).wait()
            return 0

        lax.fori_loop(0, n_grp, grp, 0)
        plsc.subcore_barrier()
        pltpu.sync_copy(accum.at[pl.ds(s * ROWS_PT, ROWS_PT)],
                        out_hbm.at[pl.ds(c * N_PAD + s * ROWS_PT, ROWS_PT)])

    return deg_kernel


def _make_edge_kernel(n_ch):
    G = n_ch // NGRP
    n_outer = G // NBUF

    @functools.partial(
        pl.kernel,
        out_type=jax.ShapeDtypeStruct((NC * N_PAD, D), jnp.float32),
        mesh=_mesh,
        scratch_types=[
            pltpu.VMEM_SHARED((N_PAD, D), jnp.float32),
            pltpu.VMEM((G, CHUNK), jnp.int32),
            pltpu.VMEM((G, CHUNK), jnp.int32),
            pltpu.VMEM((NBUF, CHUNK, D), jnp.float32),
        ] + [pltpu.SemaphoreType.DMA] * (2 * NBUF + 1),
    )
    def edge_kernel(hs_hbm, src_hbm, dst_hbm, out_hbm,
                    accum, src_g, dst_g, rows,
                    g0, g1, s0, s1, wsem):
        gsem = (g0, g1)
        ssem = (s0, s1)
        c = lax.axis_index("c")
        s = lax.axis_index("s")
        wid = s * NC + c

        zero16 = jnp.zeros((16,), jnp.float32)

        def zb(i, _):
            for j in range(D // 16):
                rows[0, i, pl.ds(j * 16, 16)] = zero16
            return 0

        lax.fori_loop(0, CHUNK, zb, 0)
        for k in range(ROWS_PT // CHUNK):
            pltpu.async_copy(rows.at[0],
                             accum.at[pl.ds(s * ROWS_PT + k * CHUNK, CHUNK)],
                             wsem)
        for k in range(ROWS_PT // CHUNK):
            pltpu.make_async_copy(rows.at[0], accum.at[pl.ds(0, CHUNK)],
                                  wsem).wait()
        plsc.subcore_barrier()

        def gfire(j, b):
            pltpu.async_copy(hs_hbm.at[src_g.at[j]], rows.at[b], gsem[b])

        def gwait(b):
            pltpu.make_async_copy(hs_hbm.at[src_g.at[0]], rows.at[b],
                                  gsem[b]).wait()

        def sfire(j, b):
            pltpu.async_copy(rows.at[b], accum.at[dst_g.at[j]], ssem[b],
                             add=True)

        def swait(b):
            pltpu.make_async_copy(rows.at[b], accum.at[dst_g.at[0]],
                                  ssem[b]).wait()

        def group(g, _):
            base = wid * n_ch + g * G
            pltpu.sync_copy(src_hbm.at[pl.ds(base, G)], src_g)
            pltpu.sync_copy(dst_hbm.at[pl.ds(base, G)], dst_g)

            for b in range(NBUF):
                gfire(b, b)

            def outer(t, _):
                for b in range(NBUF):
                    gwait(b)
                    sfire(t * NBUF + b, b)
                for b in range(NBUF):
                    swait(b)

                    @pl.when(t < n_outer - 1)
                    def _():
                        gfire((t + 1) * NBUF + b, b)

                return 0

            lax.fori_loop(0, n_outer, outer, 0)
            return 0

        lax.fori_loop(0, NGRP, group, 0)
        plsc.subcore_barrier()
        for k in range(ROWS_PT // CHUNK):
            r = s * ROWS_PT + k * CHUNK
            pltpu.async_copy(accum.at[pl.ds(r, CHUNK)],
                             out_hbm.at[pl.ds(c * N_PAD + r, CHUNK)], wsem)
        for k in range(ROWS_PT // CHUNK):
            pltpu.make_async_copy(accum.at[pl.ds(0, CHUNK)],
                                  out_hbm.at[pl.ds(0, CHUNK)], wsem).wait()

    return edge_kernel


def _dinv_body(deg_ref, out_ref):
    d = deg_ref[...]
    row = lax.rsqrt(d[0:1, :] + d[1:2, :] + 1.0)
    ri = lax.broadcasted_iota(jnp.int32, (BLK, BLK), 0)
    ci = lax.broadcasted_iota(jnp.int32, (BLK, BLK), 1)
    col = jnp.sum(jnp.where(ri == ci, row, 0.0), axis=1, keepdims=True)
    out_ref[...] = col


def _k1_body(x_ref, dinv_ref, w_ref, out_ref):
    h = jnp.dot(x_ref[...], w_ref[...], preferred_element_type=jnp.float32)
    out_ref[...] = h * dinv_ref[...]


def _k2_body(s0_ref, s1_ref, hs_ref, dinv_ref, b_ref, w_ref, out_ref):
    dinv = dinv_ref[...]
    y = jnp.maximum(dinv * (s0_ref[...] + s1_ref[...] + hs_ref[...])
                    + b_ref[...], 0.0)
    out_ref[...] = jnp.dot(y, w_ref[...],
                           preferred_element_type=jnp.float32) * dinv


def _k3_body(s0_ref, s1_ref, hs_ref, dinv_ref, b_ref, w_ref, bfc_ref, out_ref):
    dinv = dinv_ref[...]
    y = jnp.maximum(dinv * (s0_ref[...] + s1_ref[...] + hs_ref[...])
                    + b_ref[...], 0.0)
    out_ref[...] = (jnp.dot(y, w_ref[...], preferred_element_type=jnp.float32)
                    + bfc_ref[...])


_row_spec = pl.BlockSpec((BLK, D), lambda i: (i, 0))
_dinv_spec = pl.BlockSpec((BLK, 1), lambda i: (i, 0))
_w_spec = pl.BlockSpec((D, D), lambda i: (0, 0))
_b_spec = pl.BlockSpec((1, D), lambda i: (0, 0))
_s0_spec = pl.BlockSpec((BLK, D), lambda i: (i, 0))
_s1_spec = pl.BlockSpec((BLK, D), lambda i: (GRID + i, 0))

_dinv_call = pl.pallas_call(
    _dinv_body,
    grid=(GRID,),
    in_specs=[pl.BlockSpec((2, BLK), lambda i: (0, i))],
    out_specs=_dinv_spec,
    out_shape=jax.ShapeDtypeStruct((N_PAD, 1), jnp.float32),
)

_k1_call = pl.pallas_call(
    _k1_body,
    grid=(GRID,),
    in_specs=[_row_spec, _dinv_spec, _w_spec],
    out_specs=_row_spec,
    out_shape=jax.ShapeDtypeStruct((N_PAD, D), jnp.float32),
)

_k2_call = pl.pallas_call(
    _k2_body,
    grid=(GRID,),
    in_specs=[_s0_spec, _s1_spec, _row_spec, _dinv_spec, _b_spec, _w_spec],
    out_specs=_row_spec,
    out_shape=jax.ShapeDtypeStruct((N_PAD, D), jnp.float32),
)

_k3_call = pl.pallas_call(
    _k3_body,
    grid=(GRID,),
    in_specs=[_s0_spec, _s1_spec, _row_spec, _dinv_spec, _b_spec, _w_spec,
              _b_spec],
    out_specs=_row_spec,
    out_shape=jax.ShapeDtypeStruct((N_PAD, D), jnp.float32),
)


def kernel(x, edge_index, W1, b1, W2, b2, Wfc, bfc):
    E = edge_index.shape[1]
    grp = NGRP * NBUF
    n_ch = pl.cdiv(E, NW * CHUNK * grp) * grp
    e_pad = n_ch * CHUNK * NW
    n_ch_deg = e_pad // (NW * DCHUNK)
    pad = e_pad - E

    src_flat = jnp.concatenate([edge_index[0], jnp.zeros((pad,), jnp.int32)])
    dst_flat = jnp.concatenate(
        [edge_index[1], jnp.full((pad,), N_PAD - 1, jnp.int32)])
    src = src_flat.reshape(-1, CHUNK)
    dst = dst_flat.reshape(-1, CHUNK)
    dst_d = dst_flat.reshape(-1, DCHUNK)
    x_pad = jnp.pad(x, ((0, N_PAD - N), (0, 0)))

    deg_kernel = _make_deg_kernel(n_ch_deg)
    edge_kernel = _make_edge_kernel(n_ch)

    deg2 = deg_kernel(dst_d).reshape(2, N_PAD)
    dinv = _dinv_call(deg2)

    b1r = b1.reshape(1, D)
    b2r = b2.reshape(1, D)
    bfcr = bfc.reshape(1, D)

    hs1 = _k1_call(x_pad, dinv, W1)
    S1 = edge_kernel(hs1, src, dst)
    hs2 = _k2_call(S1, S1, hs1, dinv, b1r, W2)
    S2 = edge_kernel(hs2, src, dst)
    out = _k3_call(S2, S2, hs2, dinv, b2r, Wfc, bfcr)
    return out[:N]
